# speedup vs baseline: 11.5199x; 11.5199x over previous
import functools

import jax
import jax.numpy as jnp
from jax.experimental import pallas as pl
from jax.experimental.pallas import tpu as pltpu

_B, _C, _H, _W = 64, 96, 14, 14
_P = 14
_NPIX = _H * _W
_N = _NPIX + _P
_K = 9
_R = 32
_C2 = 2 * _C
_EPS = 1e-5


def _phase1_body(x_ref, w_ref, b_ref, y_ref, st_ref):
    i = pl.program_id(0)
    xb = x_ref[0]
    y = jnp.dot(w_ref[...], xb, preferred_element_type=jnp.float32)
    y = y + b_ref[...]
    y_ref[0] = y
    s = jnp.sum(y, axis=1, keepdims=True)
    q = jnp.sum(y * y, axis=1, keepdims=True)
    upd = jnp.concatenate([s, q], axis=1)

    @pl.when(i == 0)
    def _():
        st_ref[...] = jnp.zeros_like(st_ref)

    st_ref[...] += upd


def _phase2_body(y1_ref, st_ref, g1_ref, be1_ref, pr_ref, dwt_ref, db_ref,
                 gp_ref, at_ref, bmt_ref, nb_ref, uwt_ref, ub_ref,
                 emax_ref, emin_ref, acc_ref, ep_ref, lrs_ref):
    i = pl.program_id(0)
    n1 = float(_B * _NPIX)
    mu = st_ref[:, 0:1] / n1
    var = st_ref[:, 1:2] / n1 - mu * mu
    sc = g1_ref[...] / jnp.sqrt(var + _EPS)
    sh = be1_ref[...] - mu * sc
    y1n = y1_ref[0] * sc + sh
    x2 = jnp.concatenate([y1n, pr_ref[...]], axis=1)
    x2t = x2.T
    lowp = jnp.dot(x2t, dwt_ref[...], preferred_element_type=jnp.float32)
    lowp = lowp + db_ref[...]
    low = 0.5 * lowp * (1.0 + jax.lax.erf(lowp * 0.7071067811865476))
    res = jnp.dot(low, gp_ref[...], preferred_element_type=jnp.float32)
    xmt = 0.8 * x2t + 0.2 * res

    rn = jnp.sum(xmt * xmt, axis=1, keepdims=True)
    xnt = xmt / jnp.maximum(jnp.sqrt(rn), 1e-12)
    xsqc = jnp.sum(xnt * xnt, axis=1, keepdims=True)
    xn = xnt.T
    xsqr = jnp.sum(xn * xn, axis=0, keepdims=True)
    gram = jnp.dot(xnt, xn, preferred_element_type=jnp.float32)
    dist = xsqc - 2.0 * gram + xsqr

    u = jnp.dot(xmt, at_ref[...], preferred_element_type=jnp.float32)
    u = u + nb_ref[...]
    v = jnp.dot(xmt, bmt_ref[...], preferred_element_type=jnp.float32)

    ci = jax.lax.broadcasted_iota(jnp.int32, (_N, _N), 1)
    mask = jnp.zeros((_N, _N), jnp.float32)
    vmax = jnp.full((_N, _C2), -jnp.inf, jnp.float32)
    vmin = jnp.full((_N, _C2), jnp.inf, jnp.float32)
    d = dist
    for _ in range(_K):
        mnv = jnp.min(d, axis=1, keepdims=True)
        sel = d == mnv
        it = jnp.min(jnp.where(sel, ci, _N), axis=1, keepdims=True)
        oh = ci == it
        ohf = oh.astype(jnp.float32)
        mask = mask + ohf
        vt = jnp.dot(ohf, v, preferred_element_type=jnp.float32)
        vmax = jnp.maximum(vmax, vt)
        vmin = jnp.minimum(vmin, vt)
        d = jnp.where(oh, jnp.inf, d)

    s_g = jnp.dot(mask, v, preferred_element_type=jnp.float32)
    q_g = jnp.dot(mask, v * v, preferred_element_type=jnp.float32)
    lr_s = jnp.dot(mask, low, preferred_element_type=jnp.float32)

    emax_ref[0] = u + vmax
    emin_ref[0] = u + vmin
    lrs_ref[0] = lr_s

    e1 = jnp.sum(_K * u + s_g, axis=0, keepdims=True)
    e2 = jnp.sum(_K * u * u + 2.0 * u * s_g + q_g, axis=0, keepdims=True)
    upd = jnp.concatenate([e1, e2], axis=0)

    @pl.when(i == 0)
    def _():
        acc_ref[...] = jnp.zeros_like(acc_ref)

    acc_ref[...] += upd

    ep = jnp.dot(lr_s * (1.0 / _K), uwt_ref[...],
                 preferred_element_type=jnp.float32) + ub_ref[...]
    ep_ref[0] = ep


def _phase3_body(emax_ref, emin_ref, acc_ref, g2_ref, be2_ref, w2t_ref,
                 b2_ref, y3_ref, st3_ref):
    i = pl.program_id(0)
    ne = float(_B * _N * _K)
    mu = acc_ref[0:1, :] / ne
    var = acc_ref[1:2, :] / ne - mu * mu
    sc = g2_ref[...] / jnp.sqrt(var + _EPS)
    sh = be2_ref[...] - mu * sc
    z = jnp.where(sc >= 0.0, emax_ref[0], emin_ref[0])
    g = jnp.maximum(z * sc + sh, 0.0)
    y3 = jnp.dot(g, w2t_ref[...], preferred_element_type=jnp.float32)
    y3 = y3 + b2_ref[...]
    y3_ref[0] = y3
    s = jnp.sum(y3, axis=0, keepdims=True)
    q = jnp.sum(y3 * y3, axis=0, keepdims=True)
    upd = jnp.concatenate([s, q], axis=0)

    @pl.when(pl.program_id(0) == 0)
    def _():
        st3_ref[...] = jnp.zeros_like(st3_ref)

    st3_ref[...] += upd


def _phase4_body(y3_ref, ep_ref, st3_ref, g3_ref, be3_ref, x_ref, o_ref):
    n3 = float(_B * _N)
    mu = st3_ref[0:1, :] / n3
    var = st3_ref[1:2, :] / n3 - mu * mu
    sc = g3_ref[...] / jnp.sqrt(var + _EPS)
    sh = be3_ref[...] - mu * sc
    o = 0.8 * (y3_ref[0] * sc + sh) + 0.2 * ep_ref[0]
    oc = o[:_NPIX, :]
    o_ref[0] = oc.T + x_ref[0]


def kernel(x, fc1_W, fc1_b, fc1_g, fc1_be, nn_W, nn_b, nn_g, nn_be,
           fc2_W, fc2_b, fc2_g, fc2_be, node_prompts, graph_prompt,
           down_W, down_b, up_W, up_b):
    f32 = jnp.float32
    xf = x.reshape(_B, _C, _NPIX)
    at = (nn_W[:, :_C] - nn_W[:, _C:]).T
    bmt = nn_W[:, _C:].T
    dwt = down_W.T
    w2t = fc2_W.T
    uwt = up_W.T

    y1, st1 = pl.pallas_call(
        _phase1_body,
        grid=(_B,),
        in_specs=[
            pl.BlockSpec((1, _C, _NPIX), lambda b: (b, 0, 0)),
            pl.BlockSpec((_C, _C), lambda b: (0, 0)),
            pl.BlockSpec((_C, 1), lambda b: (0, 0)),
        ],
        out_specs=[
            pl.BlockSpec((1, _C, _NPIX), lambda b: (b, 0, 0)),
            pl.BlockSpec((_C, 2), lambda b: (0, 0)),
        ],
        out_shape=[
            jax.ShapeDtypeStruct((_B, _C, _NPIX), f32),
            jax.ShapeDtypeStruct((_C, 2), f32),
        ],
    )(xf, fc1_W, fc1_b.reshape(_C, 1))

    emax, emin, acc_e, ep, lrs = pl.pallas_call(
        _phase2_body,
        grid=(_B,),
        in_specs=[
            pl.BlockSpec((1, _C, _NPIX), lambda b: (b, 0, 0)),
            pl.BlockSpec((_C, 2), lambda b: (0, 0)),
            pl.BlockSpec((_C, 1), lambda b: (0, 0)),
            pl.BlockSpec((_C, 1), lambda b: (0, 0)),
            pl.BlockSpec((_C, _P), lambda b: (0, 0)),
            pl.BlockSpec((_C, _R), lambda b: (0, 0)),
            pl.BlockSpec((1, _R), lambda b: (0, 0)),
            pl.BlockSpec((_R, _C), lambda b: (0, 0)),
            pl.BlockSpec((_C, _C2), lambda b: (0, 0)),
            pl.BlockSpec((_C, _C2), lambda b: (0, 0)),
            pl.BlockSpec((1, _C2), lambda b: (0, 0)),
            pl.BlockSpec((_R, _C), lambda b: (0, 0)),
            pl.BlockSpec((1, _C), lambda b: (0, 0)),
        ],
        out_specs=[
            pl.BlockSpec((1, _N, _C2), lambda b: (b, 0, 0)),
            pl.BlockSpec((1, _N, _C2), lambda b: (b, 0, 0)),
            pl.BlockSpec((2, _C2), lambda b: (0, 0)),
            pl.BlockSpec((1, _N, _C), lambda b: (b, 0, 0)),
            pl.BlockSpec((1, _N, _R), lambda b: (b, 0, 0)),
        ],
        out_shape=[
            jax.ShapeDtypeStruct((_B, _N, _C2), f32),
            jax.ShapeDtypeStruct((_B, _N, _C2), f32),
            jax.ShapeDtypeStruct((2, _C2), f32),
            jax.ShapeDtypeStruct((_B, _N, _C), f32),
            jax.ShapeDtypeStruct((_B, _N, _R), f32),
        ],
    )(y1, st1, fc1_g.reshape(_C, 1), fc1_be.reshape(_C, 1), node_prompts,
      dwt, down_b.reshape(1, _R), graph_prompt, at, bmt,
      nn_b.reshape(1, _C2), uwt, up_b.reshape(1, _C))
    del lrs

    y3, st3 = pl.pallas_call(
        _phase3_body,
        grid=(_B,),
        in_specs=[
            pl.BlockSpec((1, _N, _C2), lambda b: (b, 0, 0)),
            pl.BlockSpec((1, _N, _C2), lambda b: (b, 0, 0)),
            pl.BlockSpec((2, _C2), lambda b: (0, 0)),
            pl.BlockSpec((1, _C2), lambda b: (0, 0)),
            pl.BlockSpec((1, _C2), lambda b: (0, 0)),
            pl.BlockSpec((_C2, _C), lambda b: (0, 0)),
            pl.BlockSpec((1, _C), lambda b: (0, 0)),
        ],
        out_specs=[
            pl.BlockSpec((1, _N, _C), lambda b: (b, 0, 0)),
            pl.BlockSpec((2, _C), lambda b: (0, 0)),
        ],
        out_shape=[
            jax.ShapeDtypeStruct((_B, _N, _C), f32),
            jax.ShapeDtypeStruct((2, _C), f32),
        ],
    )(emax, emin, acc_e, nn_g.reshape(1, _C2), nn_be.reshape(1, _C2),
      w2t, fc2_b.reshape(1, _C))

    out = pl.pallas_call(
        _phase4_body,
        grid=(_B,),
        in_specs=[
            pl.BlockSpec((1, _N, _C), lambda b: (b, 0, 0)),
            pl.BlockSpec((1, _N, _C), lambda b: (b, 0, 0)),
            pl.BlockSpec((2, _C), lambda b: (0, 0)),
            pl.BlockSpec((1, _C), lambda b: (0, 0)),
            pl.BlockSpec((1, _C), lambda b: (0, 0)),
            pl.BlockSpec((1, _C, _NPIX), lambda b: (b, 0, 0)),
        ],
        out_specs=pl.BlockSpec((1, _C, _NPIX), lambda b: (b, 0, 0)),
        out_shape=jax.ShapeDtypeStruct((_B, _C, _NPIX), f32),
    )(y3, ep, st3, fc2_g.reshape(1, _C), fc2_be.reshape(1, _C), xf)

    return out.reshape(_B, _C, _H, _W)


# speedup vs baseline: 13.0296x; 1.1311x over previous
import functools

import jax
import jax.numpy as jnp
from jax.experimental import pallas as pl
from jax.experimental.pallas import tpu as pltpu

_B, _C, _H, _W = 64, 96, 14, 14
_P = 14
_NPIX = _H * _W
_N = _NPIX + _P
_K = 9
_R = 32
_C2 = 2 * _C
_EPS = 1e-5


def _phase1_body(x_ref, w_ref, b_ref, y_ref, st_ref):
    i = pl.program_id(0)
    xb = x_ref[0]
    y = jnp.dot(w_ref[...], xb, preferred_element_type=jnp.float32)
    y = y + b_ref[...]
    y_ref[0] = y
    s = jnp.sum(y, axis=1, keepdims=True)
    q = jnp.sum(y * y, axis=1, keepdims=True)
    upd = jnp.concatenate([s, q], axis=1)

    @pl.when(i == 0)
    def _():
        st_ref[...] = jnp.zeros_like(st_ref)

    st_ref[...] += upd


def _phase2_body(y1_ref, st_ref, g1_ref, be1_ref, pr_ref, dwt_ref, db_ref,
                 gp_ref, at_ref, bmt_ref, nb_ref, uwt_ref, ub_ref,
                 emax_ref, emin_ref, acc_ref, ep_ref):
    i = pl.program_id(0)
    n1 = float(_B * _NPIX)
    mu = st_ref[:, 0:1] / n1
    var = st_ref[:, 1:2] / n1 - mu * mu
    sc = g1_ref[...] / jnp.sqrt(var + _EPS)
    sh = be1_ref[...] - mu * sc
    y1n = y1_ref[0] * sc + sh
    x2 = jnp.concatenate([y1n, pr_ref[...]], axis=1)
    x2t = x2.T
    lowp = jnp.dot(x2t, dwt_ref[...], preferred_element_type=jnp.float32)
    lowp = lowp + db_ref[...]
    low = 0.5 * lowp * (1.0 + jax.lax.erf(lowp * 0.7071067811865476))
    res = jnp.dot(low, gp_ref[...], preferred_element_type=jnp.float32)
    xmt = 0.8 * x2t + 0.2 * res

    rn = jnp.sum(xmt * xmt, axis=1, keepdims=True)
    xnt = xmt / jnp.maximum(jnp.sqrt(rn), 1e-12)
    xsqc = jnp.sum(xnt * xnt, axis=1, keepdims=True)
    xn = xnt.T
    xsqr = jnp.sum(xn * xn, axis=0, keepdims=True)
    gram = jnp.dot(xnt, xn, preferred_element_type=jnp.float32)
    dist = xsqc - 2.0 * gram + xsqr

    u = jnp.dot(xmt, at_ref[...], preferred_element_type=jnp.float32)
    u = u + nb_ref[...]
    v = jnp.dot(xmt, bmt_ref[...], preferred_element_type=jnp.float32)

    cif = jax.lax.broadcasted_iota(jnp.int32, (_N, _N), 1).astype(jnp.float32)
    mask = jnp.zeros((_N, _N), jnp.float32)
    vmax = jnp.full((_N, _C2), -jnp.inf, jnp.float32)
    vmin = jnp.full((_N, _C2), jnp.inf, jnp.float32)
    d = dist
    for _ in range(_K):
        mnv = jnp.min(d, axis=1, keepdims=True)
        it = jnp.min(jnp.where(d == mnv, cif, jnp.inf), axis=1,
                     keepdims=True)
        ohf = (cif == it).astype(jnp.float32)
        mask = mask + ohf
        vt = jnp.dot(ohf, v, preferred_element_type=jnp.float32)
        vmax = jnp.maximum(vmax, vt)
        vmin = jnp.minimum(vmin, vt)
        d = jnp.where(ohf != 0.0, jnp.inf, d)

    s_g = jnp.dot(mask, v, preferred_element_type=jnp.float32)
    q_g = jnp.dot(mask, v * v, preferred_element_type=jnp.float32)
    lr_s = jnp.dot(mask, low, preferred_element_type=jnp.float32)

    emax_ref[0] = u + vmax
    emin_ref[0] = u + vmin

    e1 = jnp.sum(_K * u + s_g, axis=0, keepdims=True)
    e2 = jnp.sum(_K * u * u + 2.0 * u * s_g + q_g, axis=0, keepdims=True)
    upd = jnp.concatenate([e1, e2], axis=0)

    @pl.when(i == 0)
    def _():
        acc_ref[...] = jnp.zeros_like(acc_ref)

    acc_ref[...] += upd

    ep = jnp.dot(lr_s * (1.0 / _K), uwt_ref[...],
                 preferred_element_type=jnp.float32) + ub_ref[...]
    ep_ref[0] = ep


def _phase3_body(emax_ref, emin_ref, acc_ref, g2_ref, be2_ref, w2t_ref,
                 b2_ref, y3_ref, st3_ref):
    i = pl.program_id(0)
    ne = float(_B * _N * _K)
    mu = acc_ref[0:1, :] / ne
    var = acc_ref[1:2, :] / ne - mu * mu
    sc = g2_ref[...] / jnp.sqrt(var + _EPS)
    sh = be2_ref[...] - mu * sc
    z = jnp.where(sc >= 0.0, emax_ref[0], emin_ref[0])
    g = jnp.maximum(z * sc + sh, 0.0)
    y3 = jnp.dot(g, w2t_ref[...], preferred_element_type=jnp.float32)
    y3 = y3 + b2_ref[...]
    y3_ref[0] = y3
    s = jnp.sum(y3, axis=0, keepdims=True)
    q = jnp.sum(y3 * y3, axis=0, keepdims=True)
    upd = jnp.concatenate([s, q], axis=0)

    @pl.when(pl.program_id(0) == 0)
    def _():
        st3_ref[...] = jnp.zeros_like(st3_ref)

    st3_ref[...] += upd


def _phase4_body(y3_ref, ep_ref, st3_ref, g3_ref, be3_ref, x_ref, o_ref):
    n3 = float(_B * _N)
    mu = st3_ref[0:1, :] / n3
    var = st3_ref[1:2, :] / n3 - mu * mu
    sc = g3_ref[...] / jnp.sqrt(var + _EPS)
    sh = be3_ref[...] - mu * sc
    o = 0.8 * (y3_ref[0] * sc + sh) + 0.2 * ep_ref[0]
    oc = o[:_NPIX, :]
    o_ref[0] = oc.T + x_ref[0]


def kernel(x, fc1_W, fc1_b, fc1_g, fc1_be, nn_W, nn_b, nn_g, nn_be,
           fc2_W, fc2_b, fc2_g, fc2_be, node_prompts, graph_prompt,
           down_W, down_b, up_W, up_b):
    f32 = jnp.float32
    xf = x.reshape(_B, _C, _NPIX)
    at = (nn_W[:, :_C] - nn_W[:, _C:]).T
    bmt = nn_W[:, _C:].T
    dwt = down_W.T
    w2t = fc2_W.T
    uwt = up_W.T

    y1, st1 = pl.pallas_call(
        _phase1_body,
        grid=(_B,),
        in_specs=[
            pl.BlockSpec((1, _C, _NPIX), lambda b: (b, 0, 0)),
            pl.BlockSpec((_C, _C), lambda b: (0, 0)),
            pl.BlockSpec((_C, 1), lambda b: (0, 0)),
        ],
        out_specs=[
            pl.BlockSpec((1, _C, _NPIX), lambda b: (b, 0, 0)),
            pl.BlockSpec((_C, 2), lambda b: (0, 0)),
        ],
        out_shape=[
            jax.ShapeDtypeStruct((_B, _C, _NPIX), f32),
            jax.ShapeDtypeStruct((_C, 2), f32),
        ],
    )(xf, fc1_W, fc1_b.reshape(_C, 1))

    emax, emin, acc_e, ep = pl.pallas_call(
        _phase2_body,
        grid=(_B,),
        in_specs=[
            pl.BlockSpec((1, _C, _NPIX), lambda b: (b, 0, 0)),
            pl.BlockSpec((_C, 2), lambda b: (0, 0)),
            pl.BlockSpec((_C, 1), lambda b: (0, 0)),
            pl.BlockSpec((_C, 1), lambda b: (0, 0)),
            pl.BlockSpec((_C, _P), lambda b: (0, 0)),
            pl.BlockSpec((_C, _R), lambda b: (0, 0)),
            pl.BlockSpec((1, _R), lambda b: (0, 0)),
            pl.BlockSpec((_R, _C), lambda b: (0, 0)),
            pl.BlockSpec((_C, _C2), lambda b: (0, 0)),
            pl.BlockSpec((_C, _C2), lambda b: (0, 0)),
            pl.BlockSpec((1, _C2), lambda b: (0, 0)),
            pl.BlockSpec((_R, _C), lambda b: (0, 0)),
            pl.BlockSpec((1, _C), lambda b: (0, 0)),
        ],
        out_specs=[
            pl.BlockSpec((1, _N, _C2), lambda b: (b, 0, 0)),
            pl.BlockSpec((1, _N, _C2), lambda b: (b, 0, 0)),
            pl.BlockSpec((2, _C2), lambda b: (0, 0)),
            pl.BlockSpec((1, _N, _C), lambda b: (b, 0, 0)),
        ],
        out_shape=[
            jax.ShapeDtypeStruct((_B, _N, _C2), f32),
            jax.ShapeDtypeStruct((_B, _N, _C2), f32),
            jax.ShapeDtypeStruct((2, _C2), f32),
            jax.ShapeDtypeStruct((_B, _N, _C), f32),
        ],
    )(y1, st1, fc1_g.reshape(_C, 1), fc1_be.reshape(_C, 1), node_prompts,
      dwt, down_b.reshape(1, _R), graph_prompt, at, bmt,
      nn_b.reshape(1, _C2), uwt, up_b.reshape(1, _C))

    y3, st3 = pl.pallas_call(
        _phase3_body,
        grid=(_B,),
        in_specs=[
            pl.BlockSpec((1, _N, _C2), lambda b: (b, 0, 0)),
            pl.BlockSpec((1, _N, _C2), lambda b: (b, 0, 0)),
            pl.BlockSpec((2, _C2), lambda b: (0, 0)),
            pl.BlockSpec((1, _C2), lambda b: (0, 0)),
            pl.BlockSpec((1, _C2), lambda b: (0, 0)),
            pl.BlockSpec((_C2, _C), lambda b: (0, 0)),
            pl.BlockSpec((1, _C), lambda b: (0, 0)),
        ],
        out_specs=[
            pl.BlockSpec((1, _N, _C), lambda b: (b, 0, 0)),
            pl.BlockSpec((2, _C), lambda b: (0, 0)),
        ],
        out_shape=[
            jax.ShapeDtypeStruct((_B, _N, _C), f32),
            jax.ShapeDtypeStruct((2, _C), f32),
        ],
    )(emax, emin, acc_e, nn_g.reshape(1, _C2), nn_be.reshape(1, _C2),
      w2t, fc2_b.reshape(1, _C))

    out = pl.pallas_call(
        _phase4_body,
        grid=(_B,),
        in_specs=[
            pl.BlockSpec((1, _N, _C), lambda b: (b, 0, 0)),
            pl.BlockSpec((1, _N, _C), lambda b: (b, 0, 0)),
            pl.BlockSpec((2, _C), lambda b: (0, 0)),
            pl.BlockSpec((1, _C), lambda b: (0, 0)),
            pl.BlockSpec((1, _C), lambda b: (0, 0)),
            pl.BlockSpec((1, _C, _NPIX), lambda b: (b, 0, 0)),
        ],
        out_specs=pl.BlockSpec((1, _C, _NPIX), lambda b: (b, 0, 0)),
        out_shape=jax.ShapeDtypeStruct((_B, _C, _NPIX), f32),
    )(y3, ep, st3, fc2_g.reshape(1, _C), fc2_be.reshape(1, _C), xf)

    return out.reshape(_B, _C, _H, _W)


# speedup vs baseline: 18.8950x; 1.4502x over previous
import functools

import jax
import jax.numpy as jnp
from jax.experimental import pallas as pl
from jax.experimental.pallas import tpu as pltpu

_B, _C, _H, _W = 64, 96, 14, 14
_P = 14
_NPIX = _H * _W
_N = _NPIX + _P
_K = 9
_R = 32
_C2 = 2 * _C
_EPS = 1e-5
_NB = 4
_G = _B // _NB


def _phase1_body(x_ref, w_ref, b_ref, y_ref, st_ref):
    i = pl.program_id(0)
    upd = jnp.zeros((_C, 2), jnp.float32)
    for s in range(_NB):
        y = jnp.dot(w_ref[...], x_ref[s], preferred_element_type=jnp.float32)
        y = y + b_ref[...]
        y_ref[s] = y
        sm = jnp.sum(y, axis=1, keepdims=True)
        sq = jnp.sum(y * y, axis=1, keepdims=True)
        upd = upd + jnp.concatenate([sm, sq], axis=1)

    @pl.when(i == 0)
    def _():
        st_ref[...] = jnp.zeros_like(st_ref)

    st_ref[...] += upd


def _phase2_one(y1, sc, sh, pr, dwt, db, gp, at, bmt, nb, uwt, ub):
    y1n = y1 * sc + sh
    x2 = jnp.concatenate([y1n, pr], axis=1)
    x2t = x2.T
    lowp = jnp.dot(x2t, dwt, preferred_element_type=jnp.float32)
    lowp = lowp + db
    low = 0.5 * lowp * (1.0 + jax.lax.erf(lowp * 0.7071067811865476))
    res = jnp.dot(low, gp, preferred_element_type=jnp.float32)
    xmt = 0.8 * x2t + 0.2 * res

    rn = jnp.sum(xmt * xmt, axis=1, keepdims=True)
    xnt = xmt / jnp.maximum(jnp.sqrt(rn), 1e-12)
    xsqc = jnp.sum(xnt * xnt, axis=1, keepdims=True)
    xn = xnt.T
    xsqr = jnp.sum(xn * xn, axis=0, keepdims=True)
    gram = jnp.dot(xnt, xn, preferred_element_type=jnp.float32)
    dist = xsqc - 2.0 * gram + xsqr

    u = jnp.dot(xmt, at, preferred_element_type=jnp.float32)
    u = u + nb
    v = jnp.dot(xmt, bmt, preferred_element_type=jnp.float32)

    cif = jax.lax.broadcasted_iota(jnp.int32, (_N, _N), 1).astype(jnp.float32)
    mask = jnp.zeros((_N, _N), jnp.float32)
    vmax = jnp.full((_N, _C2), -jnp.inf, jnp.float32)
    vmin = jnp.full((_N, _C2), jnp.inf, jnp.float32)
    d = dist
    for _ in range(_K):
        mnv = jnp.min(d, axis=1, keepdims=True)
        it = jnp.min(jnp.where(d == mnv, cif, jnp.inf), axis=1,
                     keepdims=True)
        ohf = (cif == it).astype(jnp.float32)
        mask = mask + ohf
        vt = jnp.dot(ohf, v, preferred_element_type=jnp.float32)
        vmax = jnp.maximum(vmax, vt)
        vmin = jnp.minimum(vmin, vt)
        d = jnp.where(ohf != 0.0, jnp.inf, d)

    s_g = jnp.dot(mask, v, preferred_element_type=jnp.float32)
    q_g = jnp.dot(mask, v * v, preferred_element_type=jnp.float32)
    lr_s = jnp.dot(mask, low, preferred_element_type=jnp.float32)

    e1 = jnp.sum(_K * u + s_g, axis=0, keepdims=True)
    e2 = jnp.sum(_K * u * u + 2.0 * u * s_g + q_g, axis=0, keepdims=True)
    upd = jnp.concatenate([e1, e2], axis=0)

    ep = jnp.dot(lr_s * (1.0 / _K), uwt,
                 preferred_element_type=jnp.float32) + ub
    return u + vmax, u + vmin, upd, ep


def _phase2_body(y1_ref, st_ref, g1_ref, be1_ref, pr_ref, dwt_ref, db_ref,
                 gp_ref, at_ref, bmt_ref, nb_ref, uwt_ref, ub_ref,
                 emax_ref, emin_ref, acc_ref, ep_ref):
    i = pl.program_id(0)
    n1 = float(_B * _NPIX)
    mu = st_ref[:, 0:1] / n1
    var = st_ref[:, 1:2] / n1 - mu * mu
    sc = g1_ref[...] / jnp.sqrt(var + _EPS)
    sh = be1_ref[...] - mu * sc
    acc = jnp.zeros((2, _C2), jnp.float32)
    for s in range(_NB):
        emax, emin, upd, ep = _phase2_one(
            y1_ref[s], sc, sh, pr_ref[...], dwt_ref[...], db_ref[...],
            gp_ref[...], at_ref[...], bmt_ref[...], nb_ref[...],
            uwt_ref[...], ub_ref[...])
        emax_ref[s] = emax
        emin_ref[s] = emin
        ep_ref[s] = ep
        acc = acc + upd

    @pl.when(i == 0)
    def _():
        acc_ref[...] = jnp.zeros_like(acc_ref)

    acc_ref[...] += acc


def _phase3_body(emax_ref, emin_ref, acc_ref, g2_ref, be2_ref, w2t_ref,
                 b2_ref, y3_ref, st3_ref):
    i = pl.program_id(0)
    ne = float(_B * _N * _K)
    mu = acc_ref[0:1, :] / ne
    var = acc_ref[1:2, :] / ne - mu * mu
    sc = g2_ref[...] / jnp.sqrt(var + _EPS)
    sh = be2_ref[...] - mu * sc
    upd = jnp.zeros((2, _C), jnp.float32)
    for s in range(_NB):
        z = jnp.where(sc >= 0.0, emax_ref[s], emin_ref[s])
        g = jnp.maximum(z * sc + sh, 0.0)
        y3 = jnp.dot(g, w2t_ref[...], preferred_element_type=jnp.float32)
        y3 = y3 + b2_ref[...]
        y3_ref[s] = y3
        sm = jnp.sum(y3, axis=0, keepdims=True)
        sq = jnp.sum(y3 * y3, axis=0, keepdims=True)
        upd = upd + jnp.concatenate([sm, sq], axis=0)

    @pl.when(i == 0)
    def _():
        st3_ref[...] = jnp.zeros_like(st3_ref)

    st3_ref[...] += upd


def _phase4_body(y3_ref, ep_ref, st3_ref, g3_ref, be3_ref, x_ref, o_ref):
    n3 = float(_B * _N)
    mu = st3_ref[0:1, :] / n3
    var = st3_ref[1:2, :] / n3 - mu * mu
    sc = g3_ref[...] / jnp.sqrt(var + _EPS)
    sh = be3_ref[...] - mu * sc
    for s in range(_NB):
        o = 0.8 * (y3_ref[s] * sc + sh) + 0.2 * ep_ref[s]
        oc = o[:_NPIX, :]
        o_ref[s] = oc.T + x_ref[s]


def kernel(x, fc1_W, fc1_b, fc1_g, fc1_be, nn_W, nn_b, nn_g, nn_be,
           fc2_W, fc2_b, fc2_g, fc2_be, node_prompts, graph_prompt,
           down_W, down_b, up_W, up_b):
    f32 = jnp.float32
    xf = x.reshape(_B, _C, _NPIX)
    at = (nn_W[:, :_C] - nn_W[:, _C:]).T
    bmt = nn_W[:, _C:].T
    dwt = down_W.T
    w2t = fc2_W.T
    uwt = up_W.T

    y1, st1 = pl.pallas_call(
        _phase1_body,
        grid=(_G,),
        in_specs=[
            pl.BlockSpec((_NB, _C, _NPIX), lambda b: (b, 0, 0)),
            pl.BlockSpec((_C, _C), lambda b: (0, 0)),
            pl.BlockSpec((_C, 1), lambda b: (0, 0)),
        ],
        out_specs=[
            pl.BlockSpec((_NB, _C, _NPIX), lambda b: (b, 0, 0)),
            pl.BlockSpec((_C, 2), lambda b: (0, 0)),
        ],
        out_shape=[
            jax.ShapeDtypeStruct((_B, _C, _NPIX), f32),
            jax.ShapeDtypeStruct((_C, 2), f32),
        ],
    )(xf, fc1_W, fc1_b.reshape(_C, 1))

    emax, emin, acc_e, ep = pl.pallas_call(
        _phase2_body,
        grid=(_G,),
        in_specs=[
            pl.BlockSpec((_NB, _C, _NPIX), lambda b: (b, 0, 0)),
            pl.BlockSpec((_C, 2), lambda b: (0, 0)),
            pl.BlockSpec((_C, 1), lambda b: (0, 0)),
            pl.BlockSpec((_C, 1), lambda b: (0, 0)),
            pl.BlockSpec((_C, _P), lambda b: (0, 0)),
            pl.BlockSpec((_C, _R), lambda b: (0, 0)),
            pl.BlockSpec((1, _R), lambda b: (0, 0)),
            pl.BlockSpec((_R, _C), lambda b: (0, 0)),
            pl.BlockSpec((_C, _C2), lambda b: (0, 0)),
            pl.BlockSpec((_C, _C2), lambda b: (0, 0)),
            pl.BlockSpec((1, _C2), lambda b: (0, 0)),
            pl.BlockSpec((_R, _C), lambda b: (0, 0)),
            pl.BlockSpec((1, _C), lambda b: (0, 0)),
        ],
        out_specs=[
            pl.BlockSpec((_NB, _N, _C2), lambda b: (b, 0, 0)),
            pl.BlockSpec((_NB, _N, _C2), lambda b: (b, 0, 0)),
            pl.BlockSpec((2, _C2), lambda b: (0, 0)),
            pl.BlockSpec((_NB, _N, _C), lambda b: (b, 0, 0)),
        ],
        out_shape=[
            jax.ShapeDtypeStruct((_B, _N, _C2), f32),
            jax.ShapeDtypeStruct((_B, _N, _C2), f32),
            jax.ShapeDtypeStruct((2, _C2), f32),
            jax.ShapeDtypeStruct((_B, _N, _C), f32),
        ],
    )(y1, st1, fc1_g.reshape(_C, 1), fc1_be.reshape(_C, 1), node_prompts,
      dwt, down_b.reshape(1, _R), graph_prompt, at, bmt,
      nn_b.reshape(1, _C2), uwt, up_b.reshape(1, _C))

    y3, st3 = pl.pallas_call(
        _phase3_body,
        grid=(_G,),
        in_specs=[
            pl.BlockSpec((_NB, _N, _C2), lambda b: (b, 0, 0)),
            pl.BlockSpec((_NB, _N, _C2), lambda b: (b, 0, 0)),
            pl.BlockSpec((2, _C2), lambda b: (0, 0)),
            pl.BlockSpec((1, _C2), lambda b: (0, 0)),
            pl.BlockSpec((1, _C2), lambda b: (0, 0)),
            pl.BlockSpec((_C2, _C), lambda b: (0, 0)),
            pl.BlockSpec((1, _C), lambda b: (0, 0)),
        ],
        out_specs=[
            pl.BlockSpec((_NB, _N, _C), lambda b: (b, 0, 0)),
            pl.BlockSpec((2, _C), lambda b: (0, 0)),
        ],
        out_shape=[
            jax.ShapeDtypeStruct((_B, _N, _C), f32),
            jax.ShapeDtypeStruct((2, _C), f32),
        ],
    )(emax, emin, acc_e, nn_g.reshape(1, _C2), nn_be.reshape(1, _C2),
      w2t, fc2_b.reshape(1, _C))

    out = pl.pallas_call(
        _phase4_body,
        grid=(_G,),
        in_specs=[
            pl.BlockSpec((_NB, _N, _C), lambda b: (b, 0, 0)),
            pl.BlockSpec((_NB, _N, _C), lambda b: (b, 0, 0)),
            pl.BlockSpec((2, _C), lambda b: (0, 0)),
            pl.BlockSpec((1, _C), lambda b: (0, 0)),
            pl.BlockSpec((1, _C), lambda b: (0, 0)),
            pl.BlockSpec((_NB, _C, _NPIX), lambda b: (b, 0, 0)),
        ],
        out_specs=pl.BlockSpec((_NB, _C, _NPIX), lambda b: (b, 0, 0)),
        out_shape=jax.ShapeDtypeStruct((_B, _C, _NPIX), f32),
    )(y3, ep, st3, fc2_g.reshape(1, _C), fc2_be.reshape(1, _C), xf)

    return out.reshape(_B, _C, _H, _W)


# speedup vs baseline: 20.2019x; 1.0692x over previous
import functools

import jax
import jax.numpy as jnp
from jax.experimental import pallas as pl
from jax.experimental.pallas import tpu as pltpu

_B, _C, _H, _W = 64, 96, 14, 14
_P = 14
_NPIX = _H * _W
_N = _NPIX + _P
_K = 9
_R = 32
_C2 = 2 * _C
_EPS = 1e-5
_NB = 8
_G = _B // _NB


def _phase1_body(x_ref, w_ref, b_ref, y_ref, st_ref):
    i = pl.program_id(0)
    upd = jnp.zeros((_C, 2), jnp.float32)
    for s in range(_NB):
        y = jnp.dot(w_ref[...], x_ref[s], preferred_element_type=jnp.float32)
        y = y + b_ref[...]
        y_ref[s] = y
        sm = jnp.sum(y, axis=1, keepdims=True)
        sq = jnp.sum(y * y, axis=1, keepdims=True)
        upd = upd + jnp.concatenate([sm, sq], axis=1)

    @pl.when(i == 0)
    def _():
        st_ref[...] = jnp.zeros_like(st_ref)

    st_ref[...] += upd


def _phase2_one(y1, sc, sh, pr, dwt, db, gp, at, bmt, nb, uwt, ub):
    y1n = y1 * sc + sh
    x2 = jnp.concatenate([y1n, pr], axis=1)
    x2t = x2.T
    lowp = jnp.dot(x2t, dwt, preferred_element_type=jnp.float32)
    lowp = lowp + db
    low = 0.5 * lowp * (1.0 + jax.lax.erf(lowp * 0.7071067811865476))
    res = jnp.dot(low, gp, preferred_element_type=jnp.float32)
    xmt = 0.8 * x2t + 0.2 * res

    rn = jnp.sum(xmt * xmt, axis=1, keepdims=True)
    xnt = xmt / jnp.maximum(jnp.sqrt(rn), 1e-12)
    xsqc = jnp.sum(xnt * xnt, axis=1, keepdims=True)
    xn = xnt.T
    xsqr = jnp.sum(xn * xn, axis=0, keepdims=True)
    gram = jnp.dot(xnt, xn, preferred_element_type=jnp.float32)
    dist = xsqc - 2.0 * gram + xsqr

    u = jnp.dot(xmt, at, preferred_element_type=jnp.float32)
    u = u + nb
    v = jnp.dot(xmt, bmt, preferred_element_type=jnp.float32)

    cif = jax.lax.broadcasted_iota(jnp.int32, (_N, _N), 1).astype(jnp.float32)
    mask = jnp.zeros((_N, _N), jnp.float32)
    vmax = jnp.full((_N, _C2), -jnp.inf, jnp.float32)
    vmin = jnp.full((_N, _C2), jnp.inf, jnp.float32)
    d = dist
    for _ in range(_K):
        mnv = jnp.min(d, axis=1, keepdims=True)
        it = jnp.min(jnp.where(d == mnv, cif, jnp.inf), axis=1,
                     keepdims=True)
        ohf = (cif == it).astype(jnp.float32)
        mask = mask + ohf
        vt = jnp.dot(ohf, v, preferred_element_type=jnp.float32)
        vmax = jnp.maximum(vmax, vt)
        vmin = jnp.minimum(vmin, vt)
        d = jnp.where(ohf != 0.0, jnp.inf, d)

    s_g = jnp.dot(mask, v, preferred_element_type=jnp.float32)
    q_g = jnp.dot(mask, v * v, preferred_element_type=jnp.float32)
    lr_s = jnp.dot(mask, low, preferred_element_type=jnp.float32)

    e1 = jnp.sum(_K * u + s_g, axis=0, keepdims=True)
    e2 = jnp.sum(_K * u * u + 2.0 * u * s_g + q_g, axis=0, keepdims=True)
    upd = jnp.concatenate([e1, e2], axis=0)

    ep = jnp.dot(lr_s * (1.0 / _K), uwt,
                 preferred_element_type=jnp.float32) + ub
    return u + vmax, u + vmin, upd, ep


def _phase2_body(y1_ref, st_ref, g1_ref, be1_ref, pr_ref, dwt_ref, db_ref,
                 gp_ref, at_ref, bmt_ref, nb_ref, uwt_ref, ub_ref,
                 emax_ref, emin_ref, acc_ref, ep_ref):
    i = pl.program_id(0)
    n1 = float(_B * _NPIX)
    mu = st_ref[:, 0:1] / n1
    var = st_ref[:, 1:2] / n1 - mu * mu
    sc = g1_ref[...] / jnp.sqrt(var + _EPS)
    sh = be1_ref[...] - mu * sc
    acc = jnp.zeros((2, _C2), jnp.float32)
    for s in range(_NB):
        emax, emin, upd, ep = _phase2_one(
            y1_ref[s], sc, sh, pr_ref[...], dwt_ref[...], db_ref[...],
            gp_ref[...], at_ref[...], bmt_ref[...], nb_ref[...],
            uwt_ref[...], ub_ref[...])
        emax_ref[s] = emax
        emin_ref[s] = emin
        ep_ref[s] = ep
        acc = acc + upd

    @pl.when(i == 0)
    def _():
        acc_ref[...] = jnp.zeros_like(acc_ref)

    acc_ref[...] += acc


def _phase3_body(emax_ref, emin_ref, acc_ref, g2_ref, be2_ref, w2t_ref,
                 b2_ref, y3_ref, st3_ref):
    i = pl.program_id(0)
    ne = float(_B * _N * _K)
    mu = acc_ref[0:1, :] / ne
    var = acc_ref[1:2, :] / ne - mu * mu
    sc = g2_ref[...] / jnp.sqrt(var + _EPS)
    sh = be2_ref[...] - mu * sc
    upd = jnp.zeros((2, _C), jnp.float32)
    for s in range(_NB):
        z = jnp.where(sc >= 0.0, emax_ref[s], emin_ref[s])
        g = jnp.maximum(z * sc + sh, 0.0)
        y3 = jnp.dot(g, w2t_ref[...], preferred_element_type=jnp.float32)
        y3 = y3 + b2_ref[...]
        y3_ref[s] = y3
        sm = jnp.sum(y3, axis=0, keepdims=True)
        sq = jnp.sum(y3 * y3, axis=0, keepdims=True)
        upd = upd + jnp.concatenate([sm, sq], axis=0)

    @pl.when(i == 0)
    def _():
        st3_ref[...] = jnp.zeros_like(st3_ref)

    st3_ref[...] += upd


def _phase4_body(y3_ref, ep_ref, st3_ref, g3_ref, be3_ref, x_ref, o_ref):
    n3 = float(_B * _N)
    mu = st3_ref[0:1, :] / n3
    var = st3_ref[1:2, :] / n3 - mu * mu
    sc = g3_ref[...] / jnp.sqrt(var + _EPS)
    sh = be3_ref[...] - mu * sc
    for s in range(_NB):
        o = 0.8 * (y3_ref[s] * sc + sh) + 0.2 * ep_ref[s]
        oc = o[:_NPIX, :]
        o_ref[s] = oc.T + x_ref[s]


def kernel(x, fc1_W, fc1_b, fc1_g, fc1_be, nn_W, nn_b, nn_g, nn_be,
           fc2_W, fc2_b, fc2_g, fc2_be, node_prompts, graph_prompt,
           down_W, down_b, up_W, up_b):
    f32 = jnp.float32
    xf = x.reshape(_B, _C, _NPIX)
    at = (nn_W[:, :_C] - nn_W[:, _C:]).T
    bmt = nn_W[:, _C:].T
    dwt = down_W.T
    w2t = fc2_W.T
    uwt = up_W.T

    y1, st1 = pl.pallas_call(
        _phase1_body,
        grid=(_G,),
        in_specs=[
            pl.BlockSpec((_NB, _C, _NPIX), lambda b: (b, 0, 0)),
            pl.BlockSpec((_C, _C), lambda b: (0, 0)),
            pl.BlockSpec((_C, 1), lambda b: (0, 0)),
        ],
        out_specs=[
            pl.BlockSpec((_NB, _C, _NPIX), lambda b: (b, 0, 0)),
            pl.BlockSpec((_C, 2), lambda b: (0, 0)),
        ],
        out_shape=[
            jax.ShapeDtypeStruct((_B, _C, _NPIX), f32),
            jax.ShapeDtypeStruct((_C, 2), f32),
        ],
    )(xf, fc1_W, fc1_b.reshape(_C, 1))

    emax, emin, acc_e, ep = pl.pallas_call(
        _phase2_body,
        grid=(_G,),
        in_specs=[
            pl.BlockSpec((_NB, _C, _NPIX), lambda b: (b, 0, 0)),
            pl.BlockSpec((_C, 2), lambda b: (0, 0)),
            pl.BlockSpec((_C, 1), lambda b: (0, 0)),
            pl.BlockSpec((_C, 1), lambda b: (0, 0)),
            pl.BlockSpec((_C, _P), lambda b: (0, 0)),
            pl.BlockSpec((_C, _R), lambda b: (0, 0)),
            pl.BlockSpec((1, _R), lambda b: (0, 0)),
            pl.BlockSpec((_R, _C), lambda b: (0, 0)),
            pl.BlockSpec((_C, _C2), lambda b: (0, 0)),
            pl.BlockSpec((_C, _C2), lambda b: (0, 0)),
            pl.BlockSpec((1, _C2), lambda b: (0, 0)),
            pl.BlockSpec((_R, _C), lambda b: (0, 0)),
            pl.BlockSpec((1, _C), lambda b: (0, 0)),
        ],
        out_specs=[
            pl.BlockSpec((_NB, _N, _C2), lambda b: (b, 0, 0)),
            pl.BlockSpec((_NB, _N, _C2), lambda b: (b, 0, 0)),
            pl.BlockSpec((2, _C2), lambda b: (0, 0)),
            pl.BlockSpec((_NB, _N, _C), lambda b: (b, 0, 0)),
        ],
        out_shape=[
            jax.ShapeDtypeStruct((_B, _N, _C2), f32),
            jax.ShapeDtypeStruct((_B, _N, _C2), f32),
            jax.ShapeDtypeStruct((2, _C2), f32),
            jax.ShapeDtypeStruct((_B, _N, _C), f32),
        ],
    )(y1, st1, fc1_g.reshape(_C, 1), fc1_be.reshape(_C, 1), node_prompts,
      dwt, down_b.reshape(1, _R), graph_prompt, at, bmt,
      nn_b.reshape(1, _C2), uwt, up_b.reshape(1, _C))

    y3, st3 = pl.pallas_call(
        _phase3_body,
        grid=(_G,),
        in_specs=[
            pl.BlockSpec((_NB, _N, _C2), lambda b: (b, 0, 0)),
            pl.BlockSpec((_NB, _N, _C2), lambda b: (b, 0, 0)),
            pl.BlockSpec((2, _C2), lambda b: (0, 0)),
            pl.BlockSpec((1, _C2), lambda b: (0, 0)),
            pl.BlockSpec((1, _C2), lambda b: (0, 0)),
            pl.BlockSpec((_C2, _C), lambda b: (0, 0)),
            pl.BlockSpec((1, _C), lambda b: (0, 0)),
        ],
        out_specs=[
            pl.BlockSpec((_NB, _N, _C), lambda b: (b, 0, 0)),
            pl.BlockSpec((2, _C), lambda b: (0, 0)),
        ],
        out_shape=[
            jax.ShapeDtypeStruct((_B, _N, _C), f32),
            jax.ShapeDtypeStruct((2, _C), f32),
        ],
    )(emax, emin, acc_e, nn_g.reshape(1, _C2), nn_be.reshape(1, _C2),
      w2t, fc2_b.reshape(1, _C))

    out = pl.pallas_call(
        _phase4_body,
        grid=(_G,),
        in_specs=[
            pl.BlockSpec((_NB, _N, _C), lambda b: (b, 0, 0)),
            pl.BlockSpec((_NB, _N, _C), lambda b: (b, 0, 0)),
            pl.BlockSpec((2, _C), lambda b: (0, 0)),
            pl.BlockSpec((1, _C), lambda b: (0, 0)),
            pl.BlockSpec((1, _C), lambda b: (0, 0)),
            pl.BlockSpec((_NB, _C, _NPIX), lambda b: (b, 0, 0)),
        ],
        out_specs=pl.BlockSpec((_NB, _C, _NPIX), lambda b: (b, 0, 0)),
        out_shape=jax.ShapeDtypeStruct((_B, _C, _NPIX), f32),
    )(y3, ep, st3, fc2_g.reshape(1, _C), fc2_be.reshape(1, _C), xf)

    return out.reshape(_B, _C, _H, _W)
